# Initial kernel scaffold; baseline (speedup 1.0000x reference)
import functools

import jax
import jax.numpy as jnp
from jax import lax
from jax.experimental import pallas as pl
from jax.experimental.pallas import tpu as pltpu
from jax.experimental.pallas import tpu_sc as plsc

N = 10000
D = 128
E = 320000
NC = 2
NS = 16
NW = NC * NS
EPW = E // NW
K = 80
NCH = EPW // K
RPS = N // NS
DD = 16

_mesh = plsc.VectorSubcoreMesh(
    core_axis_name="c", subcore_axis_name="s", num_cores=NC, num_subcores=NS
)


@functools.partial(
    pl.kernel,
    out_type=jax.ShapeDtypeStruct((NC, N, D), jnp.float32),
    mesh=_mesh,
    scratch_types=[
        pltpu.VMEM((K,), jnp.int32),
        pltpu.VMEM((K,), jnp.int32),
        pltpu.VMEM((K, D), jnp.float32),
        pltpu.VMEM_SHARED((N, D), jnp.float32),
        pltpu.SemaphoreType.DMA,
    ],
)
def _sc_agg(z_hbm, src_hbm, dst_hbm, zero_hbm, out_hbm, src_v, dst_v, rows_v,
            acc, sem):
    c = lax.axis_index("c")
    s = lax.axis_index("s")
    wid = c * NS + s

    pltpu.sync_copy(zero_hbm.at[pl.ds(s * RPS, RPS)],
                    acc.at[pl.ds(s * RPS, RPS)])
    plsc.subcore_barrier()

    @pl.loop(0, NCH)
    def _(i):
        base = wid * EPW + i * K
        pltpu.sync_copy(src_hbm.at[pl.ds(base, K)], src_v)
        pltpu.sync_copy(dst_hbm.at[pl.ds(base, K)], dst_v)
        pltpu.async_copy(z_hbm.at[src_v], rows_v, sem).wait()
        pltpu.sync_copy(rows_v, acc.at[dst_v], add=True)

    plsc.subcore_barrier()
    pltpu.sync_copy(acc.at[pl.ds(s * RPS, RPS)],
                    out_hbm.at[c, pl.ds(s * RPS, RPS)])


@functools.partial(
    pl.kernel,
    out_type=jax.ShapeDtypeStruct((NC, N, DD), jnp.float32),
    mesh=_mesh,
    scratch_types=[
        pltpu.VMEM((K,), jnp.int32),
        pltpu.VMEM((K, DD), jnp.float32),
        pltpu.VMEM_SHARED((N, DD), jnp.float32),
    ],
)
def _sc_deg(dst_hbm, zero_hbm, out_hbm, dst_v, ones_v, dacc):
    c = lax.axis_index("c")
    s = lax.axis_index("s")
    wid = c * NS + s

    @pl.loop(0, K)
    def _(r):
        ones_v[r, pl.ds(0, DD)] = jnp.full((DD,), 1.0, jnp.float32)

    pltpu.sync_copy(zero_hbm.at[pl.ds(s * RPS, RPS)],
                    dacc.at[pl.ds(s * RPS, RPS)])
    plsc.subcore_barrier()

    @pl.loop(0, NCH)
    def _(i):
        base = wid * EPW + i * K
        pltpu.sync_copy(dst_hbm.at[pl.ds(base, K)], dst_v)
        pltpu.sync_copy(ones_v, dacc.at[dst_v], add=True)

    plsc.subcore_barrier()
    pltpu.sync_copy(dacc.at[pl.ds(s * RPS, RPS)],
                    out_hbm.at[c, pl.ds(s * RPS, RPS)])


def _prep_body(x_ref, wl_ref, wr_ref, bl_ref, z_ref, h_ref):
    x = x_ref[...]
    z_ref[...] = jnp.dot(x, wl_ref[...], preferred_element_type=jnp.float32)
    h_ref[...] = (jnp.dot(x, wr_ref[...], preferred_element_type=jnp.float32)
                  + bl_ref[...])


_prep = pl.pallas_call(
    _prep_body,
    out_shape=[
        jax.ShapeDtypeStruct((N, D), jnp.float32),
        jax.ShapeDtypeStruct((N, D), jnp.float32),
    ],
)


def _combine_body(p_ref, degp_ref, h_ref, wl_ref, wr_ref, bl_ref, z_ref,
                  hn_ref):
    deg = degp_ref[0, :, 0:1] + degp_ref[1, :, 0:1]
    invd = 1.0 / jnp.maximum(deg, 1.0)
    x = jnp.maximum((p_ref[0] + p_ref[1]) * invd + h_ref[...], 0.0)
    z_ref[...] = jnp.dot(x, wl_ref[...], preferred_element_type=jnp.float32)
    hn_ref[...] = (jnp.dot(x, wr_ref[...], preferred_element_type=jnp.float32)
                   + bl_ref[...])


_combine = pl.pallas_call(
    _combine_body,
    out_shape=[
        jax.ShapeDtypeStruct((N, D), jnp.float32),
        jax.ShapeDtypeStruct((N, D), jnp.float32),
    ],
)


def _final_body(p_ref, degp_ref, h_ref, o_ref):
    deg = degp_ref[0, :, 0:1] + degp_ref[1, :, 0:1]
    invd = 1.0 / jnp.maximum(deg, 1.0)
    o_ref[...] = jnp.maximum((p_ref[0] + p_ref[1]) * invd + h_ref[...], 0.0)


_final = pl.pallas_call(
    _final_body,
    out_shape=jax.ShapeDtypeStruct((N, D), jnp.float32),
)


def kernel(feature, edge_index, W0l, b0l, W0r, W1l, b1l, W1r, W2l, b2l, W2r):
    src = edge_index[0]
    dst = edge_index[1]
    zeros_big = jnp.zeros((N, D), jnp.float32)
    zeros_deg = jnp.zeros((N, DD), jnp.float32)

    degp = _sc_deg(dst, zeros_deg)
    z, h = _prep(feature, W0l.T, W0r.T, b0l.reshape(1, D))
    p = _sc_agg(z, src, dst, zeros_big)
    z, h = _combine(p, degp, h, W1l.T, W1r.T, b1l.reshape(1, D))
    p = _sc_agg(z, src, dst, zeros_big)
    z, h = _combine(p, degp, h, W2l.T, W2r.T, b2l.reshape(1, D))
    p = _sc_agg(z, src, dst, zeros_big)
    return _final(p, degp, h)


# speedup vs baseline: 4.7324x; 4.7324x over previous
import functools

import jax
import jax.numpy as jnp
from jax import lax
from jax.experimental import pallas as pl
from jax.experimental.pallas import tpu as pltpu
from jax.experimental.pallas import tpu_sc as plsc

N = 10000
D = 128
E = 320000
NC = 2
NS = 16
NW = NC * NS
EPW = E // NW
K = 80
NCH = EPW // K
NP = 10240
RPS = NP // NS
DD = 16

_mesh = plsc.VectorSubcoreMesh(
    core_axis_name="c", subcore_axis_name="s", num_cores=NC, num_subcores=NS
)


@functools.partial(
    pl.kernel,
    out_type=jax.ShapeDtypeStruct((NC, NP, D), jnp.float32),
    mesh=_mesh,
    scratch_types=[
        pltpu.VMEM((K,), jnp.int32),
        pltpu.VMEM((K,), jnp.int32),
        pltpu.VMEM((K, D), jnp.float32),
        pltpu.VMEM_SHARED((NP, D), jnp.float32),
        pltpu.SemaphoreType.DMA,
    ],
)
def _sc_agg(z_hbm, src_hbm, dst_hbm, zero_hbm, out_hbm, src_v, dst_v, rows_v,
            acc, sem):
    c = lax.axis_index("c")
    s = lax.axis_index("s")
    wid = c * NS + s

    pltpu.sync_copy(zero_hbm.at[pl.ds(s * RPS, RPS)],
                    acc.at[pl.ds(s * RPS, RPS)])
    plsc.subcore_barrier()

    @pl.loop(0, NCH)
    def _(i):
        base = wid * EPW + i * K
        pltpu.sync_copy(src_hbm.at[pl.ds(base, K)], src_v)
        pltpu.sync_copy(dst_hbm.at[pl.ds(base, K)], dst_v)
        pltpu.async_copy(z_hbm.at[src_v], rows_v, sem).wait()
        pltpu.sync_copy(rows_v, acc.at[dst_v], add=True)

    plsc.subcore_barrier()
    pltpu.sync_copy(acc.at[pl.ds(s * RPS, RPS)],
                    out_hbm.at[c, pl.ds(s * RPS, RPS)])


@functools.partial(
    pl.kernel,
    out_type=jax.ShapeDtypeStruct((NC, NP, D), jnp.float32),
    mesh=_mesh,
    scratch_types=[
        pltpu.VMEM((K,), jnp.int32),
        pltpu.VMEM((K, D), jnp.float32),
        pltpu.VMEM_SHARED((NP, D), jnp.float32),
    ],
)
def _sc_deg(dst_hbm, ones_hbm, zero_hbm, out_hbm, dst_v, ones_v, dacc):
    c = lax.axis_index("c")
    s = lax.axis_index("s")
    wid = c * NS + s

    pltpu.sync_copy(ones_hbm, ones_v)
    pltpu.sync_copy(zero_hbm.at[pl.ds(s * RPS, RPS)],
                    dacc.at[pl.ds(s * RPS, RPS)])
    plsc.subcore_barrier()

    @pl.loop(0, NCH)
    def _(i):
        base = wid * EPW + i * K
        pltpu.sync_copy(dst_hbm.at[pl.ds(base, K)], dst_v)
        pltpu.sync_copy(ones_v, dacc.at[dst_v], add=True)

    plsc.subcore_barrier()
    pltpu.sync_copy(dacc.at[pl.ds(s * RPS, RPS)],
                    out_hbm.at[c, pl.ds(s * RPS, RPS)])


def _prep_body(x_ref, wl_ref, wr_ref, bl_ref, z_ref, h_ref):
    x = x_ref[...]
    z_ref[...] = jnp.dot(x, wl_ref[...], preferred_element_type=jnp.float32)
    h_ref[...] = (jnp.dot(x, wr_ref[...], preferred_element_type=jnp.float32)
                  + bl_ref[...])


_prep = pl.pallas_call(
    _prep_body,
    out_shape=[
        jax.ShapeDtypeStruct((N, D), jnp.float32),
        jax.ShapeDtypeStruct((N, D), jnp.float32),
    ],
)


def _combine_body(p_ref, degp_ref, h_ref, wl_ref, wr_ref, bl_ref, z_ref,
                  hn_ref):
    deg = degp_ref[0, 0:N, 0:1] + degp_ref[1, 0:N, 0:1]
    invd = 1.0 / jnp.maximum(deg, 1.0)
    x = jnp.maximum((p_ref[0, 0:N] + p_ref[1, 0:N]) * invd + h_ref[...], 0.0)
    z_ref[...] = jnp.dot(x, wl_ref[...], preferred_element_type=jnp.float32)
    hn_ref[...] = (jnp.dot(x, wr_ref[...], preferred_element_type=jnp.float32)
                   + bl_ref[...])


_combine = pl.pallas_call(
    _combine_body,
    out_shape=[
        jax.ShapeDtypeStruct((N, D), jnp.float32),
        jax.ShapeDtypeStruct((N, D), jnp.float32),
    ],
)


def _final_body(p_ref, degp_ref, h_ref, o_ref):
    deg = degp_ref[0, 0:N, 0:1] + degp_ref[1, 0:N, 0:1]
    invd = 1.0 / jnp.maximum(deg, 1.0)
    o_ref[...] = jnp.maximum((p_ref[0, 0:N] + p_ref[1, 0:N]) * invd + h_ref[...], 0.0)


_final = pl.pallas_call(
    _final_body,
    out_shape=jax.ShapeDtypeStruct((N, D), jnp.float32),
)


def kernel(feature, edge_index, W0l, b0l, W0r, W1l, b1l, W1r, W2l, b2l, W2r):
    src = edge_index[0]
    dst = edge_index[1]
    zeros_big = jnp.zeros((NP, D), jnp.float32)

    ones_small = jnp.ones((K, D), jnp.float32)
    degp = _sc_deg(dst, ones_small, zeros_big)
    z, h = _prep(feature, W0l.T, W0r.T, b0l.reshape(1, D))
    p = _sc_agg(z, src, dst, zeros_big)
    z, h = _combine(p, degp, h, W1l.T, W1r.T, b1l.reshape(1, D))
    p = _sc_agg(z, src, dst, zeros_big)
    z, h = _combine(p, degp, h, W2l.T, W2r.T, b2l.reshape(1, D))
    p = _sc_agg(z, src, dst, zeros_big)
    return _final(p, degp, h)


# speedup vs baseline: 10.5308x; 2.2252x over previous
import functools

import jax
import jax.numpy as jnp
from jax import lax
from jax.experimental import pallas as pl
from jax.experimental.pallas import tpu as pltpu
from jax.experimental.pallas import tpu_sc as plsc

N = 10000
D = 128
E = 320000
NC = 2
NS = 16
NW = NC * NS
EPW = E // NW
K = 80
NCH = EPW // K
NP = 10240
RPS = NP // NS
DD = 16

_mesh = plsc.VectorSubcoreMesh(
    core_axis_name="c", subcore_axis_name="s", num_cores=NC, num_subcores=NS
)


@functools.partial(
    pl.kernel,
    out_type=jax.ShapeDtypeStruct((NC, NP, D), jnp.float32),
    mesh=_mesh,
    scratch_types=[
        pltpu.VMEM((EPW,), jnp.int32),
        pltpu.VMEM((NCH, K), jnp.int32),
        pltpu.VMEM((K, D), jnp.float32),
        pltpu.VMEM((K, D), jnp.float32),
        pltpu.VMEM_SHARED((NP, D), jnp.float32),
        pltpu.SemaphoreType.DMA,
        pltpu.SemaphoreType.DMA,
    ],
)
def _sc_agg(z_hbm, src2_hbm, dst3_hbm, zero_hbm, out_hbm, src_all, dst_all,
            rows0, rows1, acc, sem0, sem1):
    c = lax.axis_index("c")
    s = lax.axis_index("s")
    wid = c * NS + s

    pltpu.sync_copy(src2_hbm.at[wid], src_all)
    pltpu.sync_copy(dst3_hbm.at[wid], dst_all)

    def _gather(i, rows, sem):
        return pltpu.async_copy(z_hbm.at[src_all.at[pl.ds(i * K, K)]],
                                rows, sem)

    _gather(0, rows0, sem0)
    _gather(1, rows1, sem1)
    pltpu.sync_copy(zero_hbm.at[pl.ds(s * RPS, RPS)],
                    acc.at[pl.ds(s * RPS, RPS)])
    plsc.subcore_barrier()

    def _gwait(rows, sem):
        pltpu.make_async_copy(z_hbm.at[src_all.at[pl.ds(0, K)]],
                              rows, sem).wait()

    @pl.loop(0, (NCH - 1) // 2)
    def _(j):
        i0 = 2 * j
        _gwait(rows0, sem0)
        pltpu.sync_copy(rows0, acc.at[dst_all.at[i0]], add=True)
        _gather(i0 + 2, rows0, sem0)

        i1 = 2 * j + 1
        _gwait(rows1, sem1)
        pltpu.sync_copy(rows1, acc.at[dst_all.at[i1]], add=True)

        @pl.when(i1 + 2 < NCH)
        def _():
            _gather(i1 + 2, rows1, sem1)

    _gwait(rows0, sem0)
    pltpu.sync_copy(rows0, acc.at[dst_all.at[NCH - 1]], add=True)

    plsc.subcore_barrier()
    pltpu.sync_copy(acc.at[pl.ds(s * RPS, RPS)],
                    out_hbm.at[c, pl.ds(s * RPS, RPS)])


@functools.partial(
    pl.kernel,
    out_type=jax.ShapeDtypeStruct((NC, NP, D), jnp.float32),
    mesh=_mesh,
    scratch_types=[
        pltpu.VMEM((NCH, K), jnp.int32),
        pltpu.VMEM((K, D), jnp.float32),
        pltpu.VMEM_SHARED((NP, D), jnp.float32),
    ],
)
def _sc_deg(dst3_hbm, ones_hbm, zero_hbm, out_hbm, dst_all, ones_v, dacc):
    c = lax.axis_index("c")
    s = lax.axis_index("s")
    wid = c * NS + s

    pltpu.sync_copy(dst3_hbm.at[wid], dst_all)
    pltpu.sync_copy(ones_hbm, ones_v)
    pltpu.sync_copy(zero_hbm.at[pl.ds(s * RPS, RPS)],
                    dacc.at[pl.ds(s * RPS, RPS)])
    plsc.subcore_barrier()

    @pl.loop(0, NCH)
    def _(i):
        pltpu.sync_copy(ones_v, dacc.at[dst_all.at[i]], add=True)

    plsc.subcore_barrier()
    pltpu.sync_copy(dacc.at[pl.ds(s * RPS, RPS)],
                    out_hbm.at[c, pl.ds(s * RPS, RPS)])


def _prep_body(x_ref, wl_ref, wr_ref, bl_ref, z_ref, h_ref):
    x = x_ref[...]
    z_ref[...] = jnp.dot(x, wl_ref[...], preferred_element_type=jnp.float32)
    h_ref[...] = (jnp.dot(x, wr_ref[...], preferred_element_type=jnp.float32)
                  + bl_ref[...])


_prep = pl.pallas_call(
    _prep_body,
    out_shape=[
        jax.ShapeDtypeStruct((N, D), jnp.float32),
        jax.ShapeDtypeStruct((N, D), jnp.float32),
    ],
)


def _combine_body(p_ref, degp_ref, h_ref, wl_ref, wr_ref, bl_ref, z_ref,
                  hn_ref):
    deg = degp_ref[0, 0:N, 0:1] + degp_ref[1, 0:N, 0:1]
    invd = 1.0 / jnp.maximum(deg, 1.0)
    x = jnp.maximum((p_ref[0, 0:N] + p_ref[1, 0:N]) * invd + h_ref[...], 0.0)
    z_ref[...] = jnp.dot(x, wl_ref[...], preferred_element_type=jnp.float32)
    hn_ref[...] = (jnp.dot(x, wr_ref[...], preferred_element_type=jnp.float32)
                   + bl_ref[...])


_combine = pl.pallas_call(
    _combine_body,
    out_shape=[
        jax.ShapeDtypeStruct((N, D), jnp.float32),
        jax.ShapeDtypeStruct((N, D), jnp.float32),
    ],
)


def _final_body(p_ref, degp_ref, h_ref, o_ref):
    deg = degp_ref[0, 0:N, 0:1] + degp_ref[1, 0:N, 0:1]
    invd = 1.0 / jnp.maximum(deg, 1.0)
    o_ref[...] = jnp.maximum((p_ref[0, 0:N] + p_ref[1, 0:N]) * invd + h_ref[...], 0.0)


_final = pl.pallas_call(
    _final_body,
    out_shape=jax.ShapeDtypeStruct((N, D), jnp.float32),
)


def kernel(feature, edge_index, W0l, b0l, W0r, W1l, b1l, W1r, W2l, b2l, W2r):
    src2 = edge_index[0].reshape(NW, EPW)
    dst3 = edge_index[1].reshape(NW, NCH, K)
    zeros_big = jnp.zeros((NP, D), jnp.float32)

    ones_small = jnp.ones((K, D), jnp.float32)
    degp = _sc_deg(dst3, ones_small, zeros_big)
    z, h = _prep(feature, W0l.T, W0r.T, b0l.reshape(1, D))
    p = _sc_agg(z, src2, dst3, zeros_big)
    z, h = _combine(p, degp, h, W1l.T, W1r.T, b1l.reshape(1, D))
    p = _sc_agg(z, src2, dst3, zeros_big)
    z, h = _combine(p, degp, h, W2l.T, W2r.T, b2l.reshape(1, D))
    p = _sc_agg(z, src2, dst3, zeros_big)
    return _final(p, degp, h)


# speedup vs baseline: 12.0475x; 1.1440x over previous
import functools

import jax
import jax.numpy as jnp
from jax import lax
from jax.experimental import pallas as pl
from jax.experimental.pallas import tpu as pltpu
from jax.experimental.pallas import tpu_sc as plsc

N = 10000
D = 128
E = 320000
NC = 2
NS = 16
NW = NC * NS
EPW = E // NW
K = 80
NCH = EPW // K
NP = 10240
RPS = NP // NS
DD = 16

_mesh = plsc.VectorSubcoreMesh(
    core_axis_name="c", subcore_axis_name="s", num_cores=NC, num_subcores=NS
)


@functools.partial(
    pl.kernel,
    out_type=jax.ShapeDtypeStruct((NC, N, D), jnp.float32),
    mesh=_mesh,
    scratch_types=[
        pltpu.VMEM((K,), jnp.int32),
        pltpu.VMEM((K,), jnp.int32),
        pltpu.VMEM((K,), jnp.int32),
        pltpu.VMEM((K,), jnp.int32),
        pltpu.VMEM((K,), jnp.int32),
        pltpu.VMEM((K,), jnp.int32),
        pltpu.VMEM((K,), jnp.int32),
        pltpu.VMEM((K,), jnp.int32),
        pltpu.VMEM((K, D), jnp.float32),
        pltpu.VMEM((K, D), jnp.float32),
        pltpu.VMEM((K, D), jnp.float32),
        pltpu.VMEM((K, D), jnp.float32),
        pltpu.VMEM_SHARED((N, D), jnp.float32),
        pltpu.SemaphoreType.DMA,
        pltpu.SemaphoreType.DMA,
        pltpu.SemaphoreType.DMA,
        pltpu.SemaphoreType.DMA,
        pltpu.SemaphoreType.DMA,
        pltpu.SemaphoreType.DMA,
        pltpu.SemaphoreType.DMA,
        pltpu.SemaphoreType.DMA,
        pltpu.SemaphoreType.DMA,
        pltpu.SemaphoreType.DMA,
        pltpu.SemaphoreType.DMA,
        pltpu.SemaphoreType.DMA,
    ],
)
def _sc_agg(z_hbm, srcR_hbm, dstR_hbm, zero_hbm, dep_hbm, out_hbm,
            sb0, sb1, sb2, sb3, db0, db1, db2, db3,
            rows0, rows1, rows2, rows3, acc,
            gA, gB, gC, gD, iA, iB, iC, iD, sA, sB, sC, sD):
    del dep_hbm
    c = lax.axis_index("c")
    s = lax.axis_index("s")
    wid = c * NS + s
    srcb = [sb0, sb1, sb2, sb3]
    dstb = [db0, db1, db2, db3]
    rowbufs = [rows0, rows1, rows2, rows3]
    gsems = [gA, gB, gC, gD]
    isems = [iA, iB, iC, iD]
    ssems = [sA, sB, sC, sD]

    def _idx(i, b):
        g = wid * NCH + i
        pltpu.async_copy(srcR_hbm.at[g, 0], srcb[b], isems[b])
        pltpu.async_copy(dstR_hbm.at[g, 0], dstb[b], isems[b])

    def _iwait(b):
        pltpu.make_async_copy(srcR_hbm.at[0, 0], srcb[b], isems[b]).wait()
        pltpu.make_async_copy(dstR_hbm.at[0, 0], dstb[b], isems[b]).wait()

    def _gather(b):
        pltpu.async_copy(z_hbm.at[srcb[b]], rowbufs[b], gsems[b])

    def _gwait(b):
        pltpu.make_async_copy(z_hbm.at[srcb[0]], rowbufs[b],
                              gsems[b]).wait()

    def _scat(b):
        pltpu.async_copy(rowbufs[b], acc.at[dstb[b]], ssems[b], add=True)

    def _swait(b):
        pltpu.make_async_copy(rowbufs[b], acc.at[dstb[b]],
                              ssems[b]).wait()

    def _initslice(src_or_acc_pair):
        pass

    for b in range(3):
        _idx(b, b)
    for b in range(2):
        _iwait(b)
        _gather(b)

    @pl.when(s < NS - 1)
    def _():
        pltpu.sync_copy(zero_hbm.at[pl.ds(s * 640, 640)],
                        acc.at[pl.ds(s * 640, 640)])

    @pl.when(s == NS - 1)
    def _():
        pltpu.sync_copy(zero_hbm.at[pl.ds(9600, 400)],
                        acc.at[pl.ds(9600, 400)])

    plsc.subcore_barrier()

    @pl.loop(0, NCH // 4)
    def _(j):
        for b in range(4):
            i = 4 * j + b
            _gwait(b)
            _scat(b)
            if b == 0:
                @pl.when(j > 0)
                def _():
                    _swait(3)
            else:
                _swait(b - 1)

            @pl.when(i + 3 < NCH)
            def _():
                _idx(i + 3, (b + 3) % 4)

            @pl.when(i + 2 < NCH)
            def _():
                _iwait((b + 2) % 4)
                _gather((b + 2) % 4)

    _gwait(0)
    _scat(0)
    _swait(3)
    _swait(0)

    plsc.subcore_barrier()

    @pl.when(s < NS - 1)
    def _():
        pltpu.sync_copy(acc.at[pl.ds(s * 640, 640)],
                        out_hbm.at[c, pl.ds(s * 640, 640)])

    @pl.when(s == NS - 1)
    def _():
        pltpu.sync_copy(acc.at[pl.ds(9600, 400)],
                        out_hbm.at[c, pl.ds(9600, 400)])


@functools.partial(
    pl.kernel,
    out_type=jax.ShapeDtypeStruct((NC, N, D), jnp.float32),
    mesh=_mesh,
    scratch_types=[
        pltpu.VMEM((NCH, K), jnp.int32),
        pltpu.VMEM((K, D), jnp.float32),
        pltpu.VMEM_SHARED((N, D), jnp.float32),
    ],
)
def _sc_deg(dst3_hbm, ones_hbm, zero_hbm, out_hbm, dst_all, ones_v, dacc):
    c = lax.axis_index("c")
    s = lax.axis_index("s")
    wid = c * NS + s

    pltpu.sync_copy(dst3_hbm.at[wid], dst_all)
    pltpu.sync_copy(ones_hbm, ones_v)

    @pl.when(s < NS - 1)
    def _():
        pltpu.sync_copy(zero_hbm.at[pl.ds(s * 640, 640)],
                        dacc.at[pl.ds(s * 640, 640)])

    @pl.when(s == NS - 1)
    def _():
        pltpu.sync_copy(zero_hbm.at[pl.ds(9600, 400)],
                        dacc.at[pl.ds(9600, 400)])

    plsc.subcore_barrier()

    @pl.loop(0, NCH)
    def _(i):
        pltpu.sync_copy(ones_v, dacc.at[dst_all.at[i]], add=True)

    plsc.subcore_barrier()

    @pl.when(s < NS - 1)
    def _():
        pltpu.sync_copy(dacc.at[pl.ds(s * 640, 640)],
                        out_hbm.at[c, pl.ds(s * 640, 640)])

    @pl.when(s == NS - 1)
    def _():
        pltpu.sync_copy(dacc.at[pl.ds(9600, 400)],
                        out_hbm.at[c, pl.ds(9600, 400)])


def _prep_body(x_ref, wl_ref, wr_ref, bl_ref, z_ref, h_ref):
    x = x_ref[...]
    z_ref[...] = jnp.dot(x, wl_ref[...], preferred_element_type=jnp.float32)
    h_ref[...] = (jnp.dot(x, wr_ref[...], preferred_element_type=jnp.float32)
                  + bl_ref[...])


_prep = pl.pallas_call(
    _prep_body,
    out_shape=[
        jax.ShapeDtypeStruct((N, D), jnp.float32),
        jax.ShapeDtypeStruct((N, D), jnp.float32),
    ],
)


def _combine_body(p_ref, degp_ref, h_ref, wl_ref, wr_ref, bl_ref, z_ref,
                  hn_ref):
    deg = degp_ref[0, :, 0:1] + degp_ref[1, :, 0:1]
    invd = 1.0 / jnp.maximum(deg, 1.0)
    x = jnp.maximum((p_ref[0] + p_ref[1]) * invd + h_ref[...], 0.0)
    z_ref[...] = jnp.dot(x, wl_ref[...], preferred_element_type=jnp.float32)
    hn_ref[...] = (jnp.dot(x, wr_ref[...], preferred_element_type=jnp.float32)
                   + bl_ref[...])


_combine = pl.pallas_call(
    _combine_body,
    out_shape=[
        jax.ShapeDtypeStruct((N, D), jnp.float32),
        jax.ShapeDtypeStruct((N, D), jnp.float32),
    ],
)


def _final_body(p_ref, degp_ref, h_ref, o_ref):
    deg = degp_ref[0, :, 0:1] + degp_ref[1, :, 0:1]
    invd = 1.0 / jnp.maximum(deg, 1.0)
    o_ref[...] = jnp.maximum((p_ref[0] + p_ref[1]) * invd + h_ref[...], 0.0)


_final = pl.pallas_call(
    _final_body,
    out_shape=jax.ShapeDtypeStruct((N, D), jnp.float32),
)


def kernel(feature, edge_index, W0l, b0l, W0r, W1l, b1l, W1r, W2l, b2l, W2r):
    srcR = edge_index[0].reshape(NW * NCH, 1, K)
    dstR = edge_index[1].reshape(NW * NCH, 1, K)
    dst3 = edge_index[1].reshape(NW, NCH, K)
    zeros_big = jnp.zeros((N, D), jnp.float32)

    ones_small = jnp.ones((K, D), jnp.float32)
    degp = _sc_deg(dst3, ones_small, zeros_big)
    z, h = _prep(feature, W0l.T, W0r.T, b0l.reshape(1, D))
    p = _sc_agg(z, srcR, dstR, zeros_big, degp)
    z, h = _combine(p, degp, h, W1l.T, W1r.T, b1l.reshape(1, D))
    p = _sc_agg(z, srcR, dstR, zeros_big, p)
    z, h = _combine(p, degp, h, W2l.T, W2r.T, b2l.reshape(1, D))
    p = _sc_agg(z, srcR, dstR, zeros_big, p)
    return _final(p, degp, h)
